# baseline (speedup 1.0000x reference)
import functools

import jax
import jax.numpy as jnp
from jax import lax
from jax.experimental import pallas as pl
from jax.experimental.pallas import tpu as pltpu
from jax.experimental.pallas import tpu_sc as plsc

_N = 10000
_E = 320000
_D = 128
_G = 64
_NC = 2
_NS = 16
_NW = _NC * _NS
_EPT = _E // _NW
_CH = 64
_FULL = _EPT // _CH
_TAIL = _EPT - _FULL * _CH
_NBUF = 4
_NGRP = _FULL // _NBUF
_RPT = 632
_RPT_LAST = _N - _RPT * (_NS - 1)

_PREC = lax.Precision.HIGHEST


_ZW = 16
_NZ = (_RPT - 8) // _ZW
_NZ_LAST = (_RPT_LAST - 8) // _ZW


def _sc_agg(m, src, dst):
    mesh = plsc.VectorSubcoreMesh(core_axis_name="c", subcore_axis_name="s")

    @functools.partial(
        pl.kernel,
        out_type=jax.ShapeDtypeStruct((_NC, _N, _D), jnp.float32),
        mesh=mesh,
        scratch_types=[
            pltpu.VMEM((_EPT,), jnp.int32),
            pltpu.VMEM((_NBUF, _CH), jnp.int32),
            pltpu.VMEM((_NBUF, _CH, _D), jnp.float32),
            pltpu.VMEM((_TAIL, _D), jnp.float32),
            pltpu.VMEM((_TAIL,), jnp.int32),
            pltpu.VMEM_SHARED((_N, _D), jnp.float32),
            pltpu.SemaphoreType.DMA,
            pltpu.SemaphoreType.DMA,
            pltpu.SemaphoreType.DMA,
            pltpu.SemaphoreType.DMA,
            pltpu.SemaphoreType.DMA,
            pltpu.SemaphoreType.DMA,
            pltpu.SemaphoreType.DMA,
            pltpu.SemaphoreType.DMA,
            pltpu.SemaphoreType.DMA,
        ],
    )
    def k(m_hbm, src_hbm, dst_hbm, out_hbm, src_v, dchunk, rows,
          trows, tdst, acc_sh, zsem, g0, g1, g2, g3, s0, s1, s2, s3):
        gsem = (g0, g1, g2, g3)
        ssem = (s0, s1, s2, s3)
        cid = lax.axis_index("c")
        sid = lax.axis_index("s")
        wid = sid * _NC + cid
        e0 = wid * _EPT
        r0 = sid * _RPT
        nz = jnp.where(sid < _NS - 1, _NZ, _NZ_LAST)

        zvec = jnp.zeros((16,), jnp.float32)
        for zr in range(_ZW):
            for zc in range(_D // 16):
                trows[zr, pl.ds(zc * 16, 16)] = zvec

        @pl.loop(0, nz)
        def _(t):
            pltpu.async_copy(trows, acc_sh.at[pl.ds(r0 + t * _ZW, _ZW)],
                             zsem)

        def zrem(rpt):
            pltpu.async_copy(trows.at[pl.ds(0, 8)],
                             acc_sh.at[pl.ds(r0 + rpt - 8, 8)], zsem)

        @pl.when(sid < _NS - 1)
        def _():
            zrem(_RPT)

        @pl.when(sid == _NS - 1)
        def _():
            zrem(_RPT_LAST)

        pltpu.sync_copy(src_hbm.at[pl.ds(e0, _EPT)], src_v)

        def fire(i, b):
            pltpu.async_copy(dst_hbm.at[pl.ds(e0 + i * _CH, _CH)],
                             dchunk.at[b], gsem[b])
            pltpu.async_copy(m_hbm.at[src_v.at[pl.ds(i * _CH, _CH)]],
                             rows.at[b], gsem[b])

        def wait_gather(i, b):
            pltpu.make_async_copy(dst_hbm.at[pl.ds(e0 + i * _CH, _CH)],
                                  dchunk.at[b], gsem[b]).wait()
            pltpu.make_async_copy(m_hbm.at[src_v.at[pl.ds(i * _CH, _CH)]],
                                  rows.at[b], gsem[b]).wait()

        def wait_scatter(b):
            pltpu.make_async_copy(rows.at[b], acc_sh.at[dchunk.at[b]],
                                  ssem[b]).wait()

        for b in range(_NBUF):
            fire(b, b)

        @pl.loop(0, nz)
        def _(t):
            pltpu.make_async_copy(trows,
                                  acc_sh.at[pl.ds(r0 + t * _ZW, _ZW)],
                                  zsem).wait()

        @pl.when(sid < _NS - 1)
        def _():
            pltpu.make_async_copy(trows.at[pl.ds(0, 8)],
                                  acc_sh.at[pl.ds(r0 + _RPT - 8, 8)],
                                  zsem).wait()

        @pl.when(sid == _NS - 1)
        def _():
            pltpu.make_async_copy(trows.at[pl.ds(0, 8)],
                                  acc_sh.at[pl.ds(r0 + _RPT_LAST - 8, 8)],
                                  zsem).wait()

        plsc.subcore_barrier()

        @pl.loop(0, _NGRP)
        def _(j):
            for b in range(_NBUF):
                i = j * _NBUF + b
                wait_gather(i, b)
                pltpu.async_copy(rows.at[b], acc_sh.at[dchunk.at[b]],
                                 ssem[b], add=True)

                @pl.when(j < _NGRP - 1)
                def _(i=i, b=b):
                    wait_scatter(b)
                    fire(i + _NBUF, b)

        for b in range(_NBUF):
            wait_scatter(b)

        tbase = _FULL * _CH
        pltpu.sync_copy(dst_hbm.at[pl.ds(e0 + tbase, _TAIL)], tdst)
        pltpu.sync_copy(m_hbm.at[src_v.at[pl.ds(tbase, _TAIL)]], trows)
        pltpu.sync_copy(trows, acc_sh.at[tdst], add=True)

        plsc.subcore_barrier()

        @pl.when(sid < _NS - 1)
        def _():
            pltpu.sync_copy(acc_sh.at[pl.ds(r0, _RPT)],
                            out_hbm.at[cid].at[pl.ds(r0, _RPT)])

        @pl.when(sid == _NS - 1)
        def _():
            pltpu.sync_copy(acc_sh.at[pl.ds(r0, _RPT_LAST)],
                            out_hbm.at[cid].at[pl.ds(r0, _RPT_LAST)])

    return k(m, src, dst)


def _tc_root(h, W_root):
    def body(h_ref, w_ref, o_ref):
        o_ref[...] = jnp.dot(h_ref[...], w_ref[...],
                             preferred_element_type=jnp.float32)
    return pl.pallas_call(
        body,
        out_shape=jax.ShapeDtypeStruct((_N, _D), jnp.float32),
    )(h, W_root)


def _tc_combine(p, r, W_rel, b2d, W_root_next):
    def body(p_ref, r_ref, wl_ref, b_ref, wn_ref, ho_ref, rn_ref):
        agg = p_ref[0] + p_ref[1]
        h = (jnp.dot(agg, wl_ref[...], preferred_element_type=jnp.float32)
             + b_ref[...]
             + r_ref[...])
        ho_ref[...] = h
        rn_ref[...] = jnp.dot(h, wn_ref[...],
                              preferred_element_type=jnp.float32)
    return pl.pallas_call(
        body,
        out_shape=(jax.ShapeDtypeStruct((_N, _D), jnp.float32),
                   jax.ShapeDtypeStruct((_N, _D), jnp.float32)),
    )(p, r, W_rel, b2d, W_root_next)


def _tc_final(p, r, W_rel, b2d, batch2d, W_fc, bfc2d):
    def body(p_ref, r_ref, wl_ref, b_ref, bt_ref, wf_ref, bf_ref,
             o_ref):
        agg = p_ref[0] + p_ref[1]
        h3 = (jnp.dot(agg, wl_ref[...], preferred_element_type=jnp.float32)
              + b_ref[...]
              + r_ref[...])
        onehot = (lax.broadcasted_iota(jnp.int32, (_G, _N), 0)
                  == bt_ref[...]).astype(jnp.float32)
        sums = jnp.dot(onehot, h3, preferred_element_type=jnp.float32,
                       precision=_PREC)
        counts = jnp.sum(onehot, axis=1, keepdims=True)
        pooled = sums / jnp.maximum(counts, 1.0)
        logits = jnp.dot(pooled, wf_ref[...],
                         preferred_element_type=jnp.float32) + bf_ref[...]
        o_ref[...] = jax.nn.log_softmax(logits, axis=1)
    return pl.pallas_call(
        body,
        out_shape=jax.ShapeDtypeStruct((_G, 10), jnp.float32),
    )(p, r, W_rel, b2d, batch2d, W_fc, bfc2d)


def kernel(x, edge_index, batch,
           W_rel0, b_rel0, W_root0,
           W_rel1, b_rel1, W_root1,
           W_rel2, b_rel2, W_root2,
           W_fc, b_fc):
    src = edge_index[0]
    dst = edge_index[1]
    batch2d = batch.reshape(1, _N)

    p0 = _sc_agg(x, src, dst)
    r0 = _tc_root(x, W_root0)
    h1, r1 = _tc_combine(p0, r0, W_rel0, b_rel0.reshape(1, _D), W_root1)
    p1 = _sc_agg(h1, src, dst)
    h2, r2 = _tc_combine(p1, r1, W_rel1, b_rel1.reshape(1, _D), W_root2)
    p2 = _sc_agg(h2, src, dst)
    return _tc_final(p2, r2, W_rel2, b_rel2.reshape(1, _D),
                     batch2d, W_fc, b_fc.reshape(1, 10))


# speedup vs baseline: 1.0053x; 1.0053x over previous
import functools

import jax
import jax.numpy as jnp
from jax import lax
from jax.experimental import pallas as pl
from jax.experimental.pallas import tpu as pltpu
from jax.experimental.pallas import tpu_sc as plsc

_N = 10000
_E = 320000
_D = 128
_G = 64
_NC = 2
_NS = 16
_NW = _NC * _NS
_EPT = _E // _NW
_CH = 32
_FULL = _EPT // _CH
_TAIL = _EPT - _FULL * _CH
_NBUF = 8
_NGRP = _FULL // _NBUF
_RPT = 632
_RPT_LAST = _N - _RPT * (_NS - 1)

_PREC = lax.Precision.HIGHEST


_ZW = 16
_NZ = (_RPT - 8) // _ZW
_NZ_LAST = (_RPT_LAST - 8) // _ZW


def _sc_agg(m, src, dst):
    mesh = plsc.VectorSubcoreMesh(core_axis_name="c", subcore_axis_name="s")

    @functools.partial(
        pl.kernel,
        out_type=jax.ShapeDtypeStruct((_NC, _N, _D), jnp.float32),
        mesh=mesh,
        scratch_types=[
            pltpu.VMEM((_EPT,), jnp.int32),
            pltpu.VMEM((_NBUF, _CH), jnp.int32),
            pltpu.VMEM((_NBUF, _CH, _D), jnp.float32),
            pltpu.VMEM((_TAIL, _D), jnp.float32),
            pltpu.VMEM((_TAIL,), jnp.int32),
            pltpu.VMEM_SHARED((_N, _D), jnp.float32),
            pltpu.SemaphoreType.DMA,
            pltpu.SemaphoreType.DMA,
            pltpu.SemaphoreType.DMA,
            pltpu.SemaphoreType.DMA,
            pltpu.SemaphoreType.DMA,
            pltpu.SemaphoreType.DMA,
            pltpu.SemaphoreType.DMA,
            pltpu.SemaphoreType.DMA,
            pltpu.SemaphoreType.DMA,
            pltpu.SemaphoreType.DMA,
            pltpu.SemaphoreType.DMA,
            pltpu.SemaphoreType.DMA,
            pltpu.SemaphoreType.DMA,
            pltpu.SemaphoreType.DMA,
            pltpu.SemaphoreType.DMA,
            pltpu.SemaphoreType.DMA,
            pltpu.SemaphoreType.DMA,
        ],
    )
    def k(m_hbm, src_hbm, dst_hbm, out_hbm, src_v, dchunk, rows,
          trows, tdst, acc_sh, zsem, g0, g1, g2, g3, g4, g5, g6, g7,
          s0, s1, s2, s3, s4, s5, s6, s7):
        gsem = (g0, g1, g2, g3, g4, g5, g6, g7)
        ssem = (s0, s1, s2, s3, s4, s5, s6, s7)
        cid = lax.axis_index("c")
        sid = lax.axis_index("s")
        wid = sid * _NC + cid
        e0 = wid * _EPT
        r0 = sid * _RPT
        nz = jnp.where(sid < _NS - 1, _NZ, _NZ_LAST)

        zvec = jnp.zeros((16,), jnp.float32)
        for zr in range(_ZW):
            for zc in range(_D // 16):
                trows[zr, pl.ds(zc * 16, 16)] = zvec

        @pl.loop(0, nz)
        def _(t):
            pltpu.async_copy(trows, acc_sh.at[pl.ds(r0 + t * _ZW, _ZW)],
                             zsem)

        def zrem(rpt):
            pltpu.async_copy(trows.at[pl.ds(0, 8)],
                             acc_sh.at[pl.ds(r0 + rpt - 8, 8)], zsem)

        @pl.when(sid < _NS - 1)
        def _():
            zrem(_RPT)

        @pl.when(sid == _NS - 1)
        def _():
            zrem(_RPT_LAST)

        pltpu.sync_copy(src_hbm.at[pl.ds(e0, _EPT)], src_v)

        def fire(i, b):
            pltpu.async_copy(dst_hbm.at[pl.ds(e0 + i * _CH, _CH)],
                             dchunk.at[b], gsem[b])
            pltpu.async_copy(m_hbm.at[src_v.at[pl.ds(i * _CH, _CH)]],
                             rows.at[b], gsem[b])

        def wait_gather(i, b):
            pltpu.make_async_copy(dst_hbm.at[pl.ds(e0 + i * _CH, _CH)],
                                  dchunk.at[b], gsem[b]).wait()
            pltpu.make_async_copy(m_hbm.at[src_v.at[pl.ds(i * _CH, _CH)]],
                                  rows.at[b], gsem[b]).wait()

        def wait_scatter(b):
            pltpu.make_async_copy(rows.at[b], acc_sh.at[dchunk.at[b]],
                                  ssem[b]).wait()

        for b in range(_NBUF):
            fire(b, b)

        @pl.loop(0, nz)
        def _(t):
            pltpu.make_async_copy(trows,
                                  acc_sh.at[pl.ds(r0 + t * _ZW, _ZW)],
                                  zsem).wait()

        @pl.when(sid < _NS - 1)
        def _():
            pltpu.make_async_copy(trows.at[pl.ds(0, 8)],
                                  acc_sh.at[pl.ds(r0 + _RPT - 8, 8)],
                                  zsem).wait()

        @pl.when(sid == _NS - 1)
        def _():
            pltpu.make_async_copy(trows.at[pl.ds(0, 8)],
                                  acc_sh.at[pl.ds(r0 + _RPT_LAST - 8, 8)],
                                  zsem).wait()

        plsc.subcore_barrier()

        @pl.loop(0, _NGRP)
        def _(j):
            for b in range(_NBUF):
                i = j * _NBUF + b
                wait_gather(i, b)
                pltpu.async_copy(rows.at[b], acc_sh.at[dchunk.at[b]],
                                 ssem[b], add=True)

                @pl.when(j < _NGRP - 1)
                def _(i=i, b=b):
                    wait_scatter(b)
                    fire(i + _NBUF, b)

        for b in range(_NBUF):
            wait_scatter(b)

        tbase = _FULL * _CH
        pltpu.sync_copy(dst_hbm.at[pl.ds(e0 + tbase, _TAIL)], tdst)
        pltpu.sync_copy(m_hbm.at[src_v.at[pl.ds(tbase, _TAIL)]], trows)
        pltpu.sync_copy(trows, acc_sh.at[tdst], add=True)

        plsc.subcore_barrier()

        @pl.when(sid < _NS - 1)
        def _():
            pltpu.sync_copy(acc_sh.at[pl.ds(r0, _RPT)],
                            out_hbm.at[cid].at[pl.ds(r0, _RPT)])

        @pl.when(sid == _NS - 1)
        def _():
            pltpu.sync_copy(acc_sh.at[pl.ds(r0, _RPT_LAST)],
                            out_hbm.at[cid].at[pl.ds(r0, _RPT_LAST)])

    return k(m, src, dst)


def _tc_root(h, W_root):
    def body(h_ref, w_ref, o_ref):
        o_ref[...] = jnp.dot(h_ref[...], w_ref[...],
                             preferred_element_type=jnp.float32)
    return pl.pallas_call(
        body,
        out_shape=jax.ShapeDtypeStruct((_N, _D), jnp.float32),
    )(h, W_root)


def _tc_combine(p, r, W_rel, b2d):
    def body(p_ref, r_ref, wl_ref, b_ref, ho_ref):
        agg = p_ref[0] + p_ref[1]
        ho_ref[...] = (jnp.dot(agg, wl_ref[...],
                               preferred_element_type=jnp.float32)
                       + b_ref[...]
                       + r_ref[...])
    return pl.pallas_call(
        body,
        out_shape=jax.ShapeDtypeStruct((_N, _D), jnp.float32),
    )(p, r, W_rel, b2d)


def _tc_final(p, r, W_rel, b2d, batch2d, W_fc, bfc2d):
    def body(p_ref, r_ref, wl_ref, b_ref, bt_ref, wf_ref, bf_ref,
             o_ref):
        agg = p_ref[0] + p_ref[1]
        h3 = (jnp.dot(agg, wl_ref[...], preferred_element_type=jnp.float32)
              + b_ref[...]
              + r_ref[...])
        onehot = (lax.broadcasted_iota(jnp.int32, (_G, _N), 0)
                  == bt_ref[...]).astype(jnp.float32)
        sums = jnp.dot(onehot, h3, preferred_element_type=jnp.float32,
                       precision=_PREC)
        counts = jnp.sum(onehot, axis=1, keepdims=True)
        pooled = sums / jnp.maximum(counts, 1.0)
        logits = jnp.dot(pooled, wf_ref[...],
                         preferred_element_type=jnp.float32) + bf_ref[...]
        o_ref[...] = jax.nn.log_softmax(logits, axis=1)
    return pl.pallas_call(
        body,
        out_shape=jax.ShapeDtypeStruct((_G, 10), jnp.float32),
    )(p, r, W_rel, b2d, batch2d, W_fc, bfc2d)


def kernel(x, edge_index, batch,
           W_rel0, b_rel0, W_root0,
           W_rel1, b_rel1, W_root1,
           W_rel2, b_rel2, W_root2,
           W_fc, b_fc):
    src = edge_index[0]
    dst = edge_index[1]
    batch2d = batch.reshape(1, _N)

    p0 = _sc_agg(x, src, dst)
    r0 = _tc_root(x, W_root0)
    h1 = _tc_combine(p0, r0, W_rel0, b_rel0.reshape(1, _D))
    p1 = _sc_agg(h1, src, dst)
    r1 = _tc_root(h1, W_root1)
    h2 = _tc_combine(p1, r1, W_rel1, b_rel1.reshape(1, _D))
    p2 = _sc_agg(h2, src, dst)
    r2 = _tc_root(h2, W_root2)
    return _tc_final(p2, r2, W_rel2, b_rel2.reshape(1, _D),
                     batch2d, W_fc, b_fc.reshape(1, 10))
